# Initial kernel scaffold; baseline (speedup 1.0000x reference)
import jax
import jax.numpy as jnp
from jax.experimental import pallas as pl


def kernel(xyz, W1, b1, g1, be1, W2, b2, g2, be2):
    raise NotImplementedError("write your pallas kernel here")


# speedup vs baseline: 25.7316x; 25.7316x over previous
import functools

import jax
import jax.numpy as jnp
from jax import lax
from jax.experimental import pallas as pl
from jax.experimental.pallas import tpu as pltpu
from jax.experimental.pallas import tpu_sc as plsc

_B, _N1 = 8, 4096
_S1, _K1, _R1 = 2048, 32, 0.0025
_S2, _K2, _R2 = 256, 16, 0.005
_CO1 = 128
_CO2, _CO2P = 693, 704
_D1 = 16
_D2 = 144
_EPS = 1e-5

_INTERP = False


def _fps(xs, ys, zs, n, npoint):
    b = xs.shape[0]

    def body(xs_ref, ys_ref, zs_ref, xo_ref, yo_ref, zo_ref, dist_ref):
        dist_ref[...] = jnp.full((b, n), 1e10, jnp.float32)
        iota = lax.broadcasted_iota(jnp.int32, (b, n), 1)
        lane = lax.broadcasted_iota(jnp.int32, (b, 128), 1)
        xsv = xs_ref[...]
        ysv = ys_ref[...]
        zsv = zs_ref[...]

        def outer(g, carry):
            def step(j, bufs):
                bx, by, bz = bufs
                dist = dist_ref[...]
                m = jnp.max(dist, axis=1, keepdims=True)
                idx = jnp.min(jnp.where(dist == m, iota, n), axis=1,
                              keepdims=True)
                sel = iota == idx
                cx = jnp.sum(jnp.where(sel, xsv, 0.0), axis=1, keepdims=True)
                cy = jnp.sum(jnp.where(sel, ysv, 0.0), axis=1, keepdims=True)
                cz = jnp.sum(jnp.where(sel, zsv, 0.0), axis=1, keepdims=True)
                dx = xsv - cx
                dy = ysv - cy
                dz = zsv - cz
                d = dx * dx + dy * dy + dz * dz
                dist_ref[...] = jnp.minimum(dist, d)
                sel128 = lane == j
                return (jnp.where(sel128, cx, bx),
                        jnp.where(sel128, cy, by),
                        jnp.where(sel128, cz, bz))

            z = jnp.zeros((b, 128), jnp.float32)
            bx, by, bz = lax.fori_loop(0, 128, step, (z, z, z))
            col = pl.multiple_of(g * 128, 128)
            xo_ref[:, pl.ds(col, 128)] = bx
            yo_ref[:, pl.ds(col, 128)] = by
            zo_ref[:, pl.ds(col, 128)] = bz
            return carry

        lax.fori_loop(0, npoint // 128, outer, 0)

    out = jax.ShapeDtypeStruct((b, npoint), jnp.float32)
    return pl.pallas_call(
        body,
        out_shape=(out, out, out),
        scratch_shapes=[pltpu.VMEM((b, n), jnp.float32)],
        interpret=_INTERP,
    )(xs, ys, zs)


def _ball_query(xt8, nxq8, n, s, sblk, nsample, r2, stride):
    def body(xt_ref, nx_ref, out_ref):
        big = jnp.int32(n)
        bi = pl.program_id(0)
        nx = nx_ref[0]
        xt = xt_ref[0]
        cross = jnp.dot(nx, xt, preferred_element_type=jnp.float32)
        rsq = jnp.sum(nx * nx, axis=1, keepdims=True)
        csq = jnp.sum(xt * xt, axis=0, keepdims=True)
        d = (rsq + csq) - 2.0 * cross
        iota = lax.broadcasted_iota(jnp.int32, (sblk, n), 1)
        lane = lax.broadcasted_iota(jnp.int32, (sblk, nsample), 1)
        wbase = jnp.where(d <= r2, iota, big)

        def step(k, carry):
            cur, alive, vals = carry

            def live(c, v):
                w = jnp.where(iota > c, wbase, big)
                cand = jnp.min(w, axis=1, keepdims=True)
                return (cand, jnp.any(cand != big).astype(jnp.int32),
                        jnp.where(lane == k, cand, v))

            return lax.cond(alive != 0, live,
                            lambda c, v: (c, jnp.int32(0), v), cur, vals)

        _, _, vals = lax.fori_loop(
            0, nsample, step,
            (jnp.full((sblk, 1), -1, jnp.int32), jnp.int32(1),
             jnp.full((sblk, nsample), big, jnp.int32)))
        filled = jnp.where(vals == big, vals[:, 0:1], vals)
        out_ref[0] = jnp.minimum(filled, big - 1) + bi * stride

    return pl.pallas_call(
        body,
        grid=(xt8.shape[0], s // sblk),
        in_specs=[
            pl.BlockSpec((1, 8, n), lambda bi, si: (bi, 0, 0)),
            pl.BlockSpec((1, sblk, 8), lambda bi, si: (bi, si, 0)),
        ],
        out_specs=pl.BlockSpec((1, sblk, nsample), lambda bi, si: (bi, si, 0)),
        out_shape=jax.ShapeDtypeStruct((xt8.shape[0], s, nsample), jnp.int32),
        interpret=_INTERP,
    )(xt8, nxq8)


_NC, _NS = 2, 16
_NW = _NC * _NS
_LCH = 128


def _sc_gather(table, idx2d, d):
    m = idx2d.shape[0] * _LCH
    if _INTERP:
        return table[idx2d.reshape(-1)]
    n_chunks = m // (_NW * _LCH)
    npairs = n_chunks // 2
    mesh = plsc.VectorSubcoreMesh(core_axis_name="c", subcore_axis_name="s")

    @functools.partial(
        pl.kernel,
        out_type=jax.ShapeDtypeStruct((m, d), jnp.float32),
        mesh=mesh,
        scratch_types=[
            pltpu.VMEM((n_chunks, _LCH), jnp.int32),
            pltpu.VMEM((_LCH, d), jnp.float32),
            pltpu.VMEM((_LCH, d), jnp.float32),
            pltpu.SemaphoreType.DMA,
            pltpu.SemaphoreType.DMA,
        ],
        compiler_params=pltpu.CompilerParams(use_tc_tiling_on_sc=False),
    )
    def k(table_hbm, idx_hbm, out_hbm, idx_v, buf0, buf1, sem0, sem1):
        wid = lax.axis_index("s") * _NC + lax.axis_index("c")
        rbase = wid * n_chunks
        obase = wid * n_chunks * _LCH
        pltpu.sync_copy(idx_hbm.at[pl.ds(rbase, n_chunks)], idx_v)
        pltpu.async_copy(table_hbm.at[idx_v.at[0]], buf0, sem0)

        def pair(p, carry):
            g0 = 2 * p
            pltpu.async_copy(table_hbm.at[idx_v.at[g0 + 1]], buf1, sem1)
            pltpu.make_async_copy(table_hbm.at[idx_v.at[g0]], buf0,
                                  sem0).wait()
            pltpu.sync_copy(buf0,
                            out_hbm.at[pl.ds(obase + g0 * _LCH, _LCH)])

            @pl.when(p + 1 < npairs)
            def _():
                pltpu.async_copy(table_hbm.at[idx_v.at[g0 + 2]], buf0, sem0)

            pltpu.make_async_copy(table_hbm.at[idx_v.at[g0 + 1]], buf1,
                                  sem1).wait()
            pltpu.sync_copy(buf1,
                            out_hbm.at[pl.ds(obase + (g0 + 1) * _LCH, _LCH)])
            return carry

        lax.fori_loop(0, npairs, pair, 0)

    return k(table, idx2d)


def _mlp_stats(g, nxpad, wt, bias, sch, kk, d, co):
    nb, s = nxpad.shape[0], nxpad.shape[1]

    def body(g_ref, nx_ref, w_ref, b_ref, out_ref):
        x = (g_ref[0].reshape(sch, kk, d) - nx_ref[0][:, None, :]
             ).reshape(sch * kk, d)
        h = jnp.dot(x, w_ref[...], preferred_element_type=jnp.float32) \
            + b_ref[...]

        @pl.when(jnp.logical_and(pl.program_id(0) == 0,
                                 pl.program_id(1) == 0))
        def _():
            out_ref[...] = jnp.zeros((8, co), jnp.float32)

        sm = jnp.sum(h, axis=0, keepdims=True)
        sq = jnp.sum(h * h, axis=0, keepdims=True)
        acc = jnp.concatenate([sm, sq, jnp.zeros((6, co), jnp.float32)],
                              axis=0)
        out_ref[...] = out_ref[...] + acc

    return pl.pallas_call(
        body,
        grid=(nb, s // sch),
        in_specs=[
            pl.BlockSpec((1, sch * kk, d), lambda bi, si: (bi, si, 0)),
            pl.BlockSpec((1, sch, d), lambda bi, si: (bi, si, 0)),
            pl.BlockSpec((d, co), lambda bi, si: (0, 0)),
            pl.BlockSpec((1, co), lambda bi, si: (0, 0)),
        ],
        out_specs=pl.BlockSpec((8, co), lambda bi, si: (0, 0)),
        out_shape=jax.ShapeDtypeStruct((8, co), jnp.float32),
        interpret=_INTERP,
    )(g, nxpad, wt, bias)


def _mlp_apply(g, nxpad, wt, bias, mean, gr, beta, sch, kk, d, co, dout):
    nb, s = nxpad.shape[0], nxpad.shape[1]

    def body(g_ref, nx_ref, w_ref, b_ref, mu_ref, gr_ref, be_ref, out_ref):
        nx = nx_ref[0]
        x = (g_ref[0].reshape(sch, kk, d) - nx[:, None, :]
             ).reshape(sch * kk, d)
        h = jnp.dot(x, w_ref[...], preferred_element_type=jnp.float32) \
            + b_ref[...]
        y = (h - mu_ref[...]) * gr_ref[...] + be_ref[...]
        y = jnp.maximum(y, 0.0)
        pooled = jnp.max(y.reshape(sch, kk, co), axis=1)
        if dout > co:
            out_ref[0] = jnp.concatenate(
                [nx[:, :3], pooled,
                 jnp.zeros((sch, dout - co - 3), jnp.float32)], axis=1)
        else:
            out_ref[0] = pooled

    return pl.pallas_call(
        body,
        grid=(nb, s // sch),
        in_specs=[
            pl.BlockSpec((1, sch * kk, d), lambda bi, si: (bi, si, 0)),
            pl.BlockSpec((1, sch, d), lambda bi, si: (bi, si, 0)),
            pl.BlockSpec((d, co), lambda bi, si: (0, 0)),
            pl.BlockSpec((1, co), lambda bi, si: (0, 0)),
            pl.BlockSpec((1, co), lambda bi, si: (0, 0)),
            pl.BlockSpec((1, co), lambda bi, si: (0, 0)),
            pl.BlockSpec((1, co), lambda bi, si: (0, 0)),
        ],
        out_specs=pl.BlockSpec((1, sch, dout), lambda bi, si: (bi, si, 0)),
        out_shape=jax.ShapeDtypeStruct((nb, s, dout), jnp.float32),
        interpret=_INTERP,
    )(g, nxpad, wt, bias, mean, gr, beta)


def _pad_cols(a, width):
    return jnp.pad(a, [(0, 0)] * (a.ndim - 1) + [(0, width - a.shape[-1])])


def kernel(xyz, W1, b1, g1, be1, W2, b2, g2, be2):
    ptsT = jnp.transpose(xyz[..., :3], (0, 2, 1))
    xs, ys, zs = ptsT[:, 0], ptsT[:, 1], ptsT[:, 2]

    xo, yo, zo = _fps(xs, ys, zs, _N1, _S1)
    xt8 = jnp.pad(ptsT, ((0, 0), (0, 5), (0, 0)))
    nxq = _pad_cols(jnp.stack([xo, yo, zo], axis=-1), 8)
    gidx1 = _ball_query(xt8, nxq, _N1, _S1, 256, _K1, _R1 * _R1, _N1)

    table1 = _pad_cols(xyz, _D1).reshape(_B * _N1, _D1)
    rows1 = _sc_gather(table1, gidx1.reshape(-1, _LCH), _D1)
    g1rows = rows1.reshape(_B, _S1 * _K1, _D1)
    nxpad1 = _pad_cols(jnp.stack([xo, yo, zo], axis=-1), _D1)

    w1t = _pad_cols(W1, _D1).T
    b1r = b1.reshape(1, _CO1)
    cnt1 = float(_B * _S1 * _K1)
    st1 = _mlp_stats(g1rows, nxpad1, w1t, b1r, 256, _K1, _D1, _CO1)
    mean1 = (st1[0] / cnt1).reshape(1, _CO1)
    var1 = st1[1] / cnt1 - mean1[0] * mean1[0]
    gr1 = (g1 / jnp.sqrt(var1 + _EPS)).reshape(1, _CO1)
    table2 = _mlp_apply(g1rows, nxpad1, w1t, b1r, mean1, gr1,
                        be1.reshape(1, _CO1), 256, _K1, _D1, _CO1, _D2)

    xo2, yo2, zo2 = _fps(xo, yo, zo, _S1, _S2)
    xt8b = jnp.pad(jnp.stack([xo, yo, zo], axis=1), ((0, 0), (0, 5), (0, 0)))
    nxq2 = _pad_cols(jnp.stack([xo2, yo2, zo2], axis=-1), 8)
    gidx2 = _ball_query(xt8b, nxq2, _S1, _S2, 256, _K2, _R2 * _R2, _S1)

    rows2 = _sc_gather(table2.reshape(_B * _S1, _D2),
                       gidx2.reshape(-1, _LCH), _D2)
    g2rows = rows2.reshape(_B, _S2 * _K2, _D2)
    nxpad2 = _pad_cols(jnp.stack([xo2, yo2, zo2], axis=-1), _D2)

    w2t = jnp.pad(W2.T, ((0, _D2 - W2.shape[1]), (0, _CO2P - W2.shape[0])))
    b2r = _pad_cols(b2.reshape(1, _CO2), _CO2P)
    cnt2 = float(_B * _S2 * _K2)
    st2 = _mlp_stats(g2rows, nxpad2, w2t, b2r, _S2, _K2, _D2, _CO2P)
    mean2 = (st2[0] / cnt2).reshape(1, _CO2P)
    var2 = st2[1] / cnt2 - mean2[0] * mean2[0]
    gr2 = (_pad_cols(g2, _CO2P) / jnp.sqrt(var2 + _EPS)).reshape(1, _CO2P)
    pooled2 = _mlp_apply(g2rows, nxpad2, w2t, b2r, mean2, gr2,
                         _pad_cols(be2, _CO2P).reshape(1, _CO2P),
                         _S2, _K2, _D2, _CO2P, _CO2P)

    l2_xyz = jnp.stack([xo2, yo2, zo2], axis=-1)
    return (l2_xyz, pooled2[..., :_CO2])
